# baseline (speedup 1.0000x reference)
import jax
import jax.numpy as jnp
from jax import lax
from jax.experimental import pallas as pl
from jax.experimental.pallas import tpu as pltpu
from jax.experimental.pallas import tpu_sc as plsc

NC = 2
NS = 16
CHUNK = 128


def _make_sc_agg(npad, d, nchunk):
  npt = npad // NS
  hc = nchunk // 2
  assert nchunk % 4 == 0
  mesh = plsc.VectorSubcoreMesh(
      core_axis_name="c", subcore_axis_name="s",
      num_cores=NC, num_subcores=NS)

  out_type = jax.ShapeDtypeStruct((NC, npad, d), jnp.float32)
  scratch = [
      pltpu.VMEM_SHARED((npad, d), jnp.float32),
      pltpu.VMEM((hc, CHUNK), jnp.int32),
      pltpu.VMEM((hc, CHUNK), jnp.int32),
      pltpu.VMEM((2, CHUNK, d), jnp.float32),
      pltpu.SemaphoreType.DMA,
      pltpu.SemaphoreType.DMA,
  ]

  def body(x_hbm, col_hbm, row_hbm, z_hbm, agg_out,
           agg_sh, colbuf, rowbuf, rowsv, sem0, sem1):
    c = lax.axis_index("c")
    s = lax.axis_index("s")
    wid = c * NS + s
    pltpu.sync_copy(z_hbm, agg_sh.at[pl.ds(s * npt, npt)])
    plsc.subcore_barrier()

    def gather(j, b, sem):
      return pltpu.make_async_copy(x_hbm.at[colbuf.at[j]], rowsv.at[b], sem)

    def half(h, carry):
      pltpu.sync_copy(col_hbm.at[wid].at[pl.ds(h * hc, hc)], colbuf)
      pltpu.sync_copy(row_hbm.at[wid].at[pl.ds(h * hc, hc)], rowbuf)
      gather(0, 0, sem0).start()

      def pair(g, cc):
        j0 = 2 * g
        gather(j0 + 1, 1, sem1).start()
        gather(j0, 0, sem0).wait()
        pltpu.sync_copy(rowsv.at[0], agg_sh.at[rowbuf.at[j0]], add=True)
        gather(jnp.minimum(j0 + 2, hc - 1), 0, sem0).start()
        gather(j0 + 1, 1, sem1).wait()
        pltpu.sync_copy(rowsv.at[1], agg_sh.at[rowbuf.at[j0 + 1]], add=True)
        return cc

      lax.fori_loop(0, hc // 2, pair, 0)
      gather(hc - 1, 0, sem0).wait()
      return carry

    lax.fori_loop(0, 2, half, 0)
    plsc.subcore_barrier()
    pltpu.sync_copy(agg_sh.at[pl.ds(s * npt, npt)],
                    agg_out.at[c].at[pl.ds(s * npt, npt)])

  return pl.kernel(body, out_type=out_type, mesh=mesh, scratch_types=scratch)


def _make_sc_deg(npad, dw, nchunk):
  npt = npad // NS
  mesh = plsc.VectorSubcoreMesh(
      core_axis_name="c", subcore_axis_name="s",
      num_cores=NC, num_subcores=NS)
  out_type = jax.ShapeDtypeStruct((NC, npad, dw), jnp.float32)
  scratch = [
      pltpu.VMEM_SHARED((npad, dw), jnp.float32),
      pltpu.VMEM((nchunk, CHUNK), jnp.int32),
      pltpu.VMEM((CHUNK, dw), jnp.float32),
  ]

  def body(row_hbm, z_hbm, ones_hbm, deg_out, deg_sh, rowbuf, onesv):
    c = lax.axis_index("c")
    s = lax.axis_index("s")
    wid = c * NS + s
    pltpu.sync_copy(z_hbm, deg_sh.at[pl.ds(s * npt, npt)])
    pltpu.sync_copy(ones_hbm, onesv)
    pltpu.sync_copy(row_hbm.at[wid], rowbuf)
    plsc.subcore_barrier()

    def step(j, carry):
      pltpu.sync_copy(onesv, deg_sh.at[rowbuf.at[j]], add=True)
      return carry

    lax.fori_loop(0, nchunk, step, 0)
    plsc.subcore_barrier()
    pltpu.sync_copy(deg_sh.at[pl.ds(s * npt, npt)],
                    deg_out.at[c].at[pl.ds(s * npt, npt)])

  return pl.kernel(body, out_type=out_type, mesh=mesh, scratch_types=scratch)


def _make_tc_update(npad, d, add_res):
  rblk = 1024
  assert npad % rblk == 0
  bspec = pl.BlockSpec((rblk, d), lambda i: (i, 0))

  def body(*refs):
    if add_res:
      a0, a1, dg0, dg1, w, xin, res, o = refs
    else:
      a0, a1, dg0, dg1, w, xin, o = refs
    a = a0[:, :] + a1[:, :]
    out = lax.dot_general(a, w[:, :], (((1,), (1,)), ((), ())),
                          preferred_element_type=jnp.float32)
    deg = dg0[:, 0:1] + dg1[:, 0:1]
    out = out * (1.0 / jnp.maximum(deg, 1.0)) + xin[:, :]
    if add_res:
      out = out + res[:, :]
    o[:, :] = out

  in_specs = [bspec, bspec, bspec, bspec,
              pl.BlockSpec((d, d), lambda i: (0, 0)), bspec]
  if add_res:
    in_specs.append(bspec)
  return pl.pallas_call(
      body, grid=(npad // rblk,), in_specs=in_specs, out_specs=bspec,
      out_shape=jax.ShapeDtypeStruct((npad, d), jnp.float32))


@jax.jit
def _impl(x, edge_index, W0, W1, W2):
  n, d = x.shape
  e = edge_index.shape[1]
  nw = NC * NS
  npad = ((n + NS * 64 - 1) // (NS * 64)) * (NS * 64)
  quantum = nw * CHUNK * 4
  ep = ((e + quantum - 1) // quantum) * quantum
  nchunk = ep // (nw * CHUNK)
  row_p = jnp.concatenate(
      [edge_index[0], jnp.full((ep - e,), npad - 1, jnp.int32)])
  col_p = jnp.concatenate([edge_index[1], jnp.zeros((ep - e,), jnp.int32)])
  row3 = row_p.reshape(nw, nchunk, CHUNK)
  col3 = col_p.reshape(nw, nchunk, CHUNK)
  xp = jnp.pad(x, ((0, npad - n), (0, 0)))
  npt = npad // NS
  z128 = jnp.zeros((npt, d), jnp.float32)
  ones128 = jnp.ones((CHUNK, d), jnp.float32)

  sc_agg = _make_sc_agg(npad, d, nchunk)
  sc_deg = _make_sc_deg(npad, d, nchunk)
  upd = _make_tc_update(npad, d, False)
  upd_res = _make_tc_update(npad, d, True)

  deg = sc_deg(row3, z128, ones128)
  agg = sc_agg(xp, col3, row3, z128)
  x1 = upd(agg[0], agg[1], deg[0], deg[1], W0, xp)
  agg2 = sc_agg(x1, col3, row3, z128)
  x2 = upd(agg2[0], agg2[1], deg[0], deg[1], W1, x1)
  agg3 = sc_agg(x2, col3, row3, z128)
  x3 = upd_res(agg3[0], agg3[1], deg[0], deg[1], W2, x2, xp)
  return x3[:n]


def kernel(x, edge_index, ring_polarities, W0, W1, W2):
  del ring_polarities
  return _impl(x, edge_index, W0, W1, W2)


# speedup vs baseline: 3.4494x; 3.4494x over previous
import jax
import jax.numpy as jnp
from jax import lax
from jax.experimental import pallas as pl
from jax.experimental.pallas import tpu as pltpu
from jax.experimental.pallas import tpu_sc as plsc

NC = 2
NS = 16
CHUNK = 128


def _make_sc_agg(npad, d, nchunk):
  npt = npad // NS
  hc = nchunk // 2
  assert nchunk % 4 == 0
  mesh = plsc.VectorSubcoreMesh(
      core_axis_name="c", subcore_axis_name="s",
      num_cores=NC, num_subcores=NS)

  out_type = jax.ShapeDtypeStruct((NC, npad, d), jnp.float32)
  scratch = [
      pltpu.VMEM_SHARED((npad, d), jnp.float32),
      pltpu.VMEM((hc, CHUNK), jnp.int32),
      pltpu.VMEM((hc, CHUNK), jnp.int32),
      pltpu.VMEM((2, CHUNK, d), jnp.float32),
      pltpu.SemaphoreType.DMA,
      pltpu.SemaphoreType.DMA,
  ]

  def body(x_hbm, col_hbm, row_hbm, z_hbm, agg_out,
           agg_sh, colbuf, rowbuf, rowsv, sem0, sem1):
    c = lax.axis_index("c")
    s = lax.axis_index("s")
    wid = c * NS + s
    pltpu.sync_copy(z_hbm, agg_sh.at[pl.ds(s * npt, npt)])
    plsc.subcore_barrier()

    def gather(j, b, sem):
      return pltpu.make_async_copy(x_hbm.at[colbuf.at[j]], rowsv.at[b], sem)

    def half(h, carry):
      pltpu.sync_copy(col_hbm.at[wid].at[pl.ds(h * hc, hc)], colbuf)
      pltpu.sync_copy(row_hbm.at[wid].at[pl.ds(h * hc, hc)], rowbuf)
      gather(0, 0, sem0).start()

      def pair(g, cc):
        j0 = 2 * g
        gather(j0 + 1, 1, sem1).start()
        gather(j0, 0, sem0).wait()
        pltpu.sync_copy(rowsv.at[0], agg_sh.at[rowbuf.at[j0]], add=True)
        gather(jnp.minimum(j0 + 2, hc - 1), 0, sem0).start()
        gather(j0 + 1, 1, sem1).wait()
        pltpu.sync_copy(rowsv.at[1], agg_sh.at[rowbuf.at[j0 + 1]], add=True)
        return cc

      lax.fori_loop(0, hc // 2, pair, 0)
      gather(hc - 1, 0, sem0).wait()
      return carry

    lax.fori_loop(0, 2, half, 0)
    plsc.subcore_barrier()
    pltpu.sync_copy(agg_sh.at[pl.ds(s * npt, npt)],
                    agg_out.at[c].at[pl.ds(s * npt, npt)])

  return pl.kernel(body, out_type=out_type, mesh=mesh, scratch_types=scratch)


def _make_sc_deg(npad, dw, nchunk):
  npt = npad // NS
  mesh = plsc.VectorSubcoreMesh(
      core_axis_name="c", subcore_axis_name="s",
      num_cores=NC, num_subcores=NS)
  out_type = jax.ShapeDtypeStruct((NC, npad, dw), jnp.float32)
  scratch = [
      pltpu.VMEM_SHARED((npad, dw), jnp.float32),
      pltpu.VMEM((nchunk, CHUNK), jnp.int32),
      pltpu.VMEM((CHUNK, dw), jnp.float32),
  ]

  def body(row_hbm, z_hbm, ones_hbm, deg_out, deg_sh, rowbuf, onesv):
    c = lax.axis_index("c")
    s = lax.axis_index("s")
    wid = c * NS + s
    pltpu.sync_copy(z_hbm, deg_sh.at[pl.ds(s * npt, npt)])
    pltpu.sync_copy(ones_hbm, onesv)
    pltpu.sync_copy(row_hbm.at[wid], rowbuf)
    plsc.subcore_barrier()

    def step(j, carry):
      pltpu.sync_copy(onesv, deg_sh.at[rowbuf.at[j]], add=True)
      return carry

    lax.fori_loop(0, nchunk, step, 0)
    plsc.subcore_barrier()
    pltpu.sync_copy(deg_sh.at[pl.ds(s * npt, npt)],
                    deg_out.at[c].at[pl.ds(s * npt, npt)])

  return pl.kernel(body, out_type=out_type, mesh=mesh, scratch_types=scratch)


def _make_tc_update(npad, d, add_res):
  rblk = 1024
  assert npad % rblk == 0
  bspec = pl.BlockSpec((rblk, d), lambda i: (i, 0))

  def body(*refs):
    if add_res:
      a0, a1, dg0, dg1, w, xin, res, o = refs
    else:
      a0, a1, dg0, dg1, w, xin, o = refs
    a = a0[:, :] + a1[:, :]
    out = lax.dot_general(a, w[:, :], (((1,), (1,)), ((), ())),
                          preferred_element_type=jnp.float32)
    deg = dg0[:, 0:1] + dg1[:, 0:1]
    out = out * (1.0 / jnp.maximum(deg, 1.0)) + xin[:, :]
    if add_res:
      out = out + res[:, :]
    o[:, :] = out

  in_specs = [bspec, bspec, bspec, bspec,
              pl.BlockSpec((d, d), lambda i: (0, 0)), bspec]
  if add_res:
    in_specs.append(bspec)
  return pl.pallas_call(
      body, grid=(npad // rblk,), in_specs=in_specs, out_specs=bspec,
      out_shape=jax.ShapeDtypeStruct((npad, d), jnp.float32))


@jax.jit
def _impl(x, edge_index, W0, W1, W2):
  n, d = x.shape
  e = edge_index.shape[1]
  nw = NC * NS
  npad = ((n + NS * 64 - 1) // (NS * 64)) * (NS * 64)
  quantum = nw * CHUNK * 4
  ep = ((e + quantum - 1) // quantum) * quantum
  nchunk = ep // (nw * CHUNK)
  assert npad > n
  fill = n + jnp.arange(ep - e, dtype=jnp.int32) % (npad - n)
  row_p = jnp.concatenate([edge_index[0], fill])
  col_p = jnp.concatenate([edge_index[1], fill])
  row3 = row_p.reshape(nw, nchunk, CHUNK)
  col3 = col_p.reshape(nw, nchunk, CHUNK)
  xp = jnp.pad(x, ((0, npad - n), (0, 0)))
  npt = npad // NS
  z128 = jnp.zeros((npt, d), jnp.float32)
  ones128 = jnp.ones((CHUNK, d), jnp.float32)

  sc_agg = _make_sc_agg(npad, d, nchunk)
  sc_deg = _make_sc_deg(npad, d, nchunk)
  upd = _make_tc_update(npad, d, False)
  upd_res = _make_tc_update(npad, d, True)

  deg = sc_deg(row3, z128, ones128)
  agg = sc_agg(xp, col3, row3, z128)
  x1 = upd(agg[0], agg[1], deg[0], deg[1], W0, xp)
  agg2 = sc_agg(x1, col3, row3, z128)
  x2 = upd(agg2[0], agg2[1], deg[0], deg[1], W1, x1)
  agg3 = sc_agg(x2, col3, row3, z128)
  x3 = upd_res(agg3[0], agg3[1], deg[0], deg[1], W2, x2, xp)
  return x3[:n]


def kernel(x, edge_index, ring_polarities, W0, W1, W2):
  del ring_polarities
  return _impl(x, edge_index, W0, W1, W2)
